# baseline (speedup 1.0000x reference)
import functools

import jax
import jax.numpy as jnp
from jax import lax
from jax.experimental import pallas as pl
from jax.experimental.pallas import tpu as pltpu
from jax.experimental.pallas import tpu_sc as plsc

B = 4
S = 4096
N = 2048
D = 768

NW = 32
SPW = B * N // NW
CS = 16
NCHUNK = SPW // CS
NBUF = 4
L = 16

_mesh = plsc.VectorSubcoreMesh(core_axis_name="c", subcore_axis_name="s")


@functools.partial(
    pl.kernel,
    mesh=_mesh,
    out_type=jax.ShapeDtypeStruct((B, N, 2 * D), jnp.float32),
    scratch_types=[
        pltpu.VMEM((SPW,), jnp.int32),
        pltpu.VMEM((SPW,), jnp.int32),
        pltpu.VMEM((CS, 2 * D), jnp.float32),
        pltpu.VMEM((CS, 2 * D), jnp.float32),
        pltpu.VMEM((CS, 2 * D), jnp.float32),
        pltpu.VMEM((CS, 2 * D), jnp.float32),
    ]
    + [pltpu.SemaphoreType.DMA] * 8,
)
def _sc_gather(
    table_hbm, sidx_hbm, eidx_hbm, out_hbm,
    idx_s, idx_e, buf0, buf1, buf2, buf3, *sems,
):
    wid = lax.axis_index("s") * 2 + lax.axis_index("c")
    sbase = wid * SPW
    b = sbase // N
    nb = sbase - b * N
    pltpu.sync_copy(sidx_hbm.at[pl.ds(sbase, SPW)], idx_s)
    pltpu.sync_copy(eidx_hbm.at[pl.ds(sbase, SPW)], idx_e)

    bufs = (buf0, buf1, buf2, buf3)
    gsems = sems[:4]
    wsems = sems[4:]

    def start_gathers(ci):
        k = ci % NBUF
        hs = pltpu.async_copy(
            table_hbm.at[idx_s.at[pl.ds(ci * CS, CS)]],
            bufs[k].at[:, pl.ds(0, D)],
            gsems[k],
        )
        he = pltpu.async_copy(
            table_hbm.at[idx_e.at[pl.ds(ci * CS, CS)]],
            bufs[k].at[:, pl.ds(D, D)],
            gsems[k],
        )
        return hs, he

    writes = {}
    pending = {}
    for j in range(NBUF - 1):
        pending[j] = start_gathers(j)
    for ci in range(NCHUNK):
        nxt = ci + NBUF - 1
        if nxt < NCHUNK:
            if nxt - NBUF in writes:
                writes.pop(nxt - NBUF).wait()
            pending[nxt] = start_gathers(nxt)
        cur_s, cur_e = pending.pop(ci)
        cur_s.wait()
        cur_e.wait()
        k = ci % NBUF
        writes[ci] = pltpu.async_copy(
            bufs[k], out_hbm.at[b, pl.ds(nb + ci * CS, CS), :], wsems[k]
        )
    for ci in sorted(writes):
        writes[ci].wait()


def kernel(sequence_tensor, span_indices):
    table = sequence_tensor.reshape(B * S, D)
    si = span_indices.astype(jnp.int32) + (
        jnp.arange(B, dtype=jnp.int32) * S
    )[:, None, None]
    starts = si[..., 0].reshape(B * N)
    ends = si[..., 1].reshape(B * N)
    return _sc_gather(table, starts, ends)


# speedup vs baseline: 1.0036x; 1.0036x over previous
import functools

import jax
import jax.numpy as jnp
from jax import lax
from jax.experimental import pallas as pl
from jax.experimental.pallas import tpu as pltpu
from jax.experimental.pallas import tpu_sc as plsc

B = 4
S = 4096
N = 2048
D = 768

NW = 32
SPW = B * N // NW
CS = 16
NCHUNK = SPW // CS
NBUF = 4

_mesh = plsc.VectorSubcoreMesh(core_axis_name="c", subcore_axis_name="s")


@functools.partial(
    pl.kernel,
    mesh=_mesh,
    out_type=jax.ShapeDtypeStruct((B, N, 2 * D), jnp.float32),
    scratch_types=[
        pltpu.VMEM((SPW,), jnp.int32),
        pltpu.VMEM((SPW,), jnp.int32),
        pltpu.VMEM((CS, 2 * D), jnp.float32),
        pltpu.VMEM((CS, 2 * D), jnp.float32),
        pltpu.VMEM((CS, 2 * D), jnp.float32),
        pltpu.VMEM((CS, 2 * D), jnp.float32),
    ]
    + [pltpu.SemaphoreType.DMA] * 8,
)
def _sc_gather(
    table_hbm, sidx_hbm, eidx_hbm, out_hbm,
    idx_s, idx_e, buf0, buf1, buf2, buf3, *sems,
):
    wid = lax.axis_index("s") * 2 + lax.axis_index("c")
    sbase = wid * SPW
    b = sbase // N
    nb = sbase - b * N
    pltpu.sync_copy(sidx_hbm.at[pl.ds(sbase, SPW)], idx_s)
    pltpu.sync_copy(eidx_hbm.at[pl.ds(sbase, SPW)], idx_e)

    bufs = (buf0, buf1, buf2, buf3)
    gsems = sems[:4]
    wsems = sems[4:]

    def start_gathers(ci):
        k = ci % NBUF
        hs = pltpu.async_copy(
            table_hbm.at[idx_s.at[pl.ds(ci * CS, CS)]],
            bufs[k].at[:, pl.ds(0, D)],
            gsems[k],
        )
        he = pltpu.async_copy(
            table_hbm.at[idx_e.at[pl.ds(ci * CS, CS)]],
            bufs[k].at[:, pl.ds(D, D)],
            gsems[k],
        )
        return hs, he

    writes = {}
    pending = {}
    for j in range(NBUF - 1):
        pending[j] = start_gathers(j)
    for ci in range(NCHUNK):
        nxt = ci + NBUF - 1
        if nxt < NCHUNK:
            if nxt - NBUF in writes:
                writes.pop(nxt - NBUF).wait()
            pending[nxt] = start_gathers(nxt)
        cur_s, cur_e = pending.pop(ci)
        cur_s.wait()
        cur_e.wait()
        k = ci % NBUF
        writes[ci] = pltpu.async_copy(
            bufs[k], out_hbm.at[b, pl.ds(nb + ci * CS, CS), :], wsems[k]
        )
    for ci in sorted(writes):
        writes[ci].wait()


def kernel(sequence_tensor, span_indices):
    table = sequence_tensor.reshape(B * S, D)
    si = span_indices.astype(jnp.int32) + (
        jnp.arange(B, dtype=jnp.int32) * S
    )[:, None, None]
    starts = si[..., 0].reshape(B * N)
    ends = si[..., 1].reshape(B * N)
    return _sc_gather(table, starts, ends)
